# baseline (speedup 1.0000x reference)
import functools

import jax
import jax.numpy as jnp
from jax import lax
from jax.experimental import pallas as pl
from jax.experimental.pallas import tpu as pltpu
from jax.experimental.pallas import tpu_sc as plsc

N = 10000
P = 256
V = 32
K = 27
N_PAD = 10240
NW = 32
RPW = N_PAD // NW
QW = 48
EPS = 1e-5

R_SLOW = 256
R_FAST = 384
SLOW_CORE = 0
B_FAST = 16 * R_SLOW

CH1 = 64
NC1 = RPW // CH1

KP = 28
GCH = 32
NGC = RPW // GCH
GRP = 4
NGRP = GCH // GRP
GIW = KP * GRP

BLK_A = 1000
NBLK_A = N // BLK_A
BLK_S = 1024
NBLK_S = N_PAD // BLK_S

_SC_MESH = plsc.VectorSubcoreMesh(core_axis_name="c", subcore_axis_name="s")
_SC_PARAMS = pltpu.CompilerParams(use_tc_tiling_on_sc=False)


def _tc_a_body(x_ref, wv_ref, wq_ref, xv_ref, xq_ref, m_ref, vstats_ref, acc):
    i = pl.program_id(0)
    xb = x_ref[...]
    xv = jnp.dot(xb, wv_ref[...], preferred_element_type=jnp.float32)
    xq_ref[...] = jnp.dot(xb, wq_ref[...], preferred_element_type=jnp.float32)
    xv_ref[...] = xv
    m_ref[...] = (jnp.sum(jnp.abs(xb), axis=1, keepdims=True) > 0.0).astype(
        jnp.float32)

    @pl.when(i == 0)
    def _():
        acc[...] = jnp.zeros_like(acc)

    s = jnp.sum(xv, axis=0, keepdims=True)
    ss = jnp.sum(xv * xv, axis=0, keepdims=True)
    acc[...] = acc[...] + jnp.concatenate([s, ss], axis=0)

    @pl.when(i == NBLK_A - 1)
    def _():
        vstats_ref[...] = acc[...]


def _tc_a(x, wv, wq_all):
    return pl.pallas_call(
        _tc_a_body,
        grid=(NBLK_A,),
        in_specs=[
            pl.BlockSpec((BLK_A, P), lambda i: (i, 0)),
            pl.BlockSpec((P, P), lambda i: (0, 0)),
            pl.BlockSpec((P, K * V), lambda i: (0, 0)),
        ],
        out_specs=[
            pl.BlockSpec((BLK_A, P), lambda i: (i, 0)),
            pl.BlockSpec((BLK_A, K * V), lambda i: (i, 0)),
            pl.BlockSpec((BLK_A, 1), lambda i: (i, 0)),
            pl.BlockSpec((2, P), lambda i: (0, 0)),
        ],
        out_shape=[
            jax.ShapeDtypeStruct((N, P), jnp.float32),
            jax.ShapeDtypeStruct((N, K * V), jnp.float32),
            jax.ShapeDtypeStruct((N, 1), jnp.float32),
            jax.ShapeDtypeStruct((2, P), jnp.float32),
        ],
        scratch_shapes=[pltpu.VMEM((2, P), jnp.float32)],
    )(x, wv, wq_all)


def _sc_g1_body(xq_hbm, nint_hbm, qpre_hbm, idx1, qg, acc_v, sem):
    slow = lax.axis_index("c") == SLOW_CORE
    s = lax.axis_index("s")
    base = jnp.where(slow, s * R_SLOW, B_FAST + s * R_FAST)
    nchunk = jnp.where(slow, R_SLOW // CH1, R_FAST // CH1)
    iota16 = lax.iota(jnp.int32, 16)

    def per_chunk(c, _):
        cb = base + c * CH1
        pltpu.sync_copy(nint_hbm.at[pl.ds(cb * KP, CH1 * KP)], idx1)

        def scale(t, _):
            sl = pl.ds(t * 16, 16)
            idx1[sl] = idx1[sl] * K + (iota16 + t * 16) % KP
            return 0

        lax.fori_loop(0, CH1 * KP // 16, scale, 0, unroll=4)

        def fire(g, _):
            pltpu.async_copy(xq_hbm.at[idx1.at[pl.ds(g * GIW, GIW)]],
                             qg.at[pl.ds(g * GIW, GIW)], sem)
            return 0

        lax.fori_loop(0, CH1 // GRP, fire, 0)

        def drain(g, _):
            pltpu.make_async_copy(xq_hbm.at[idx1.at[pl.ds(g * GIW, GIW)]],
                                  qg.at[pl.ds(g * GIW, GIW)], sem).wait()
            return 0

        lax.fori_loop(0, CH1 // GRP, drain, 0)

        def acc_r(r, _):
            for c2 in range(V // 16):
                sl = pl.ds(c2 * 16, 16)

                def acc_k(k, s):
                    return s + qg[r * KP + k, sl]

                acc_v[r, sl] = lax.fori_loop(1, K, acc_k, qg[r * KP, sl],
                                             unroll=2)
            return 0

        lax.fori_loop(0, CH1, acc_r, 0, unroll=2)
        pltpu.sync_copy(acc_v, qpre_hbm.at[pl.ds(cb, CH1)])
        return 0

    lax.fori_loop(0, nchunk, per_chunk, 0)


def _sc_g1(xq_flat, nin_t):
    f = functools.partial(
        pl.kernel,
        mesh=_SC_MESH,
        compiler_params=_SC_PARAMS,
        out_type=jax.ShapeDtypeStruct((N_PAD, V), jnp.float32),
        scratch_types=[
            pltpu.VMEM((CH1 * KP,), jnp.int32),
            pltpu.VMEM((CH1 * KP, V), jnp.float32),
            pltpu.VMEM((CH1, V), jnp.float32),
            pltpu.SemaphoreType.DMA,
        ],
    )(_sc_g1_body)
    return f(xq_flat, nin_t)


def _tc_c_body(qpre_ref, m_ref, g_ref, b_ref, wm_ref, qmx_ref):
    qp = qpre_ref[...]
    rows = lax.broadcasted_iota(jnp.int32, (N_PAD, 1), 0)
    valid = (rows < N).astype(jnp.float32)
    qv = qp * valid
    s = jnp.sum(qv, axis=0, keepdims=True)
    ss = jnp.sum(qv * qv, axis=0, keepdims=True)
    mean = s / float(N)
    var = ss / float(N) - mean * mean
    qf = (qp - mean) / jnp.sqrt(var + EPS) * g_ref[...] + b_ref[...]
    qf = jnp.maximum(qf, 0.0)
    qm48 = jnp.dot(qf, wm_ref[...], preferred_element_type=jnp.float32)
    mcol = (lax.broadcasted_iota(jnp.int32, (1, QW), 1) == V).astype(
        jnp.float32)
    qmx_ref[...] = qm48 + m_ref[...] * mcol


def _tc_c(q_pre, m_pad, g, b, wm48):
    return pl.pallas_call(
        _tc_c_body,
        out_shape=jax.ShapeDtypeStruct((N_PAD, QW), jnp.float32),
    )(q_pre, m_pad, g, b, wm48)


def _perm_mat(inverse):
    rows = lax.broadcasted_iota(jnp.int32, (P, P), 0)
    cols = lax.broadcasted_iota(jnp.int32, (P, P), 1)
    if inverse:
        return ((rows % V) * (P // V) + rows // V == cols).astype(jnp.float32)
    return (rows == (cols % V) * (P // V) + cols // V).astype(jnp.float32)


def _tc_c2_body(xv_ref, st_ref, g_ref, b_ref, vf_ref):
    st = st_ref[...]
    mean = st[0:1, :] / float(N)
    var = st[1:2, :] / float(N) - mean * mean
    vf = (xv_ref[...] - mean) / jnp.sqrt(var + EPS) * g_ref[...] + b_ref[...]
    vf = jnp.maximum(vf, 0.0)
    vf_ref[...] = jnp.dot(vf, _perm_mat(False),
                          preferred_element_type=jnp.float32)


def _tc_c2(xv, vstats, g, b):
    return pl.pallas_call(
        _tc_c2_body,
        grid=(NBLK_A,),
        in_specs=[
            pl.BlockSpec((BLK_A, P), lambda i: (i, 0)),
            pl.BlockSpec((2, P), lambda i: (0, 0)),
            pl.BlockSpec((1, P), lambda i: (0, 0)),
            pl.BlockSpec((1, P), lambda i: (0, 0)),
        ],
        out_specs=pl.BlockSpec((BLK_A, P), lambda i: (i, 0)),
        out_shape=jax.ShapeDtypeStruct((N, P), jnp.float32),
    )(xv, vstats, g, b)


def _sc_attn_body(qmx_hbm, vf_hbm, nint_hbm, bm_hbm, out_hbm,
                  qm_own, idx1, qxg, vg, outb, bm_v,
                  sem_q, sem_v0, sem_v1):
    slow = lax.axis_index("c") == SLOW_CORE
    s = lax.axis_index("s")
    base = jnp.where(slow, s * R_SLOW, B_FAST + s * R_FAST)
    nchunk = jnp.where(slow, R_SLOW // GCH, R_FAST // GCH)
    pltpu.sync_copy(bm_hbm, bm_v)

    def per_chunk(c, _):
        cb = base + c * GCH
        pltpu.sync_copy(qmx_hbm.at[pl.ds(cb, GCH)], qm_own)
        pltpu.sync_copy(nint_hbm.at[pl.ds(cb * KP, GCH * KP)], idx1)

        for g in range(NGRP):
            pltpu.async_copy(qmx_hbm.at[idx1.at[pl.ds(g * GIW, GIW)]],
                             qxg.at[pl.ds(g * GIW, GIW)], sem_q)
        def fire_vg(g, par):
            sem = sem_v0 if par == 0 else sem_v1
            for i in range(GIW // 16):
                pltpu.async_copy(
                    vf_hbm.at[idx1.at[pl.ds(g * GIW + i * 16, 16)]],
                    vg.at[par, pl.ds(i * 16, 16)], sem)

        def drain_vg(g, par):
            sem = sem_v0 if par == 0 else sem_v1
            for i in range(GIW // 16):
                pltpu.make_async_copy(
                    vf_hbm.at[idx1.at[pl.ds(g * GIW + i * 16, 16)]],
                    vg.at[par, pl.ds(i * 16, 16)], sem).wait()

        fire_vg(0, 0)
        fire_vg(1, 1)
        for g in range(NGRP):
            pltpu.make_async_copy(qmx_hbm.at[idx1.at[pl.ds(g * GIW, GIW)]],
                                  qxg.at[pl.ds(g * GIW, GIW)], sem_q).wait()

        def lg_r(r, _):
            def lg_k(k, _):
                p = r * KP + k
                mk = qxg[p, pl.ds(V, 16)][0]
                for c2 in range(V // 16):
                    sl = pl.ds(c2 * 16, 16)
                    qxg[p, sl] = ((qxg[p, sl] - qm_own[r, sl] * mk
                                   + bm_v[sl]) * mk)
                return 0

            lax.fori_loop(0, K, lg_k, 0, unroll=3)
            return 0

        lax.fori_loop(0, GCH, lg_r, 0)

        def sm_r(r, _):
            p0 = r * KP
            for c2 in range(V // 16):
                sl = pl.ds(c2 * 16, 16)

                def mxk(k, m):
                    return jnp.maximum(m, qxg[p0 + k, sl])

                mx = lax.fori_loop(1, K, mxk, qxg[p0, sl], unroll=2)

                def esk(k, s):
                    e = jnp.exp(qxg[p0 + k, sl] - mx)
                    qxg[p0 + k, sl] = e
                    return s + e

                s = lax.fori_loop(0, K, esk, jnp.zeros((16,), jnp.float32),
                                  unroll=3)
                rinv = 1.0 / s

                def nrm(k, _):
                    mk = qxg[p0 + k, pl.ds(V, 16)][0]
                    qxg[p0 + k, sl] = qxg[p0 + k, sl] * (rinv * mk)
                    return 0

                lax.fori_loop(0, K, nrm, 0, unroll=3)
            return 0

        lax.fori_loop(0, GCH, sm_r, 0)

        for g in range(NGRP):
            par = g % 2
            drain_vg(g, par)

            def row_j(j, _):
                r = g * GRP + j
                p0 = r * KP
                q0 = j * KP

                def k_acc(k, acc):
                    a0 = qxg[p0 + k, pl.ds(0, 16)]
                    a1 = qxg[p0 + k, pl.ds(16, 16)]
                    new = []
                    for cc in range(16):
                        row = vg[par, q0 + k, pl.ds(cc * 16, 16)]
                        new.append(acc[cc] + row * (a0 if cc % 2 == 0
                                                    else a1))
                    return tuple(new)

                acc = lax.fori_loop(
                    0, K, k_acc,
                    tuple(jnp.zeros((16,), jnp.float32) for _ in range(16)),
                    unroll=3)
                for cc in range(16):
                    outb[r, pl.ds(cc * 16, 16)] = acc[cc]
                return 0

            lax.fori_loop(0, GRP, row_j, 0)
            if g + 2 < NGRP:
                fire_vg(g + 2, par)

        pltpu.sync_copy(outb, out_hbm.at[pl.ds(cb, GCH)])
        return 0

    lax.fori_loop(0, nchunk, per_chunk, 0)


def _sc_attn(qmx, v_f, nin_t, bm):
    f = functools.partial(
        pl.kernel,
        mesh=_SC_MESH,
        compiler_params=_SC_PARAMS,
        out_type=jax.ShapeDtypeStruct((N_PAD, P), jnp.float32),
        scratch_types=[
            pltpu.VMEM((GCH, QW), jnp.float32),
            pltpu.VMEM((GCH * KP,), jnp.int32),
            pltpu.VMEM((GCH * KP, QW), jnp.float32),
            pltpu.VMEM((2, GIW, P), jnp.float32),
            pltpu.VMEM((GCH, P), jnp.float32),
            pltpu.VMEM((V,), jnp.float32),
            pltpu.SemaphoreType.DMA,
            pltpu.SemaphoreType.DMA,
            pltpu.SemaphoreType.DMA,
        ],
    )(_sc_attn_body)
    return f(qmx, v_f, nin_t, bm)


def _tc_stats_body(op_ref, st_ref, acc):
    i = pl.program_id(0)
    op = op_ref[...]
    rows = i * BLK_S + lax.broadcasted_iota(jnp.int32, (BLK_S, 1), 0)
    valid = (rows < N).astype(jnp.float32)
    ov = op * valid

    @pl.when(i == 0)
    def _():
        acc[...] = jnp.zeros_like(acc)

    s = jnp.sum(ov, axis=0, keepdims=True)
    ss = jnp.sum(ov * ov, axis=0, keepdims=True)
    acc[...] = acc[...] + jnp.concatenate([s, ss], axis=0)

    @pl.when(i == NBLK_S - 1)
    def _():
        st_ref[...] = acc[...]


def _tc_stats(out_pre):
    return pl.pallas_call(
        _tc_stats_body,
        grid=(NBLK_S,),
        in_specs=[pl.BlockSpec((BLK_S, P), lambda i: (i, 0))],
        out_specs=pl.BlockSpec((2, P), lambda i: (0, 0)),
        out_shape=jax.ShapeDtypeStruct((2, P), jnp.float32),
        scratch_shapes=[pltpu.VMEM((2, P), jnp.float32)],
    )(out_pre)


def _tc_e_body(op_ref, st_ref, g_ref, b_ref, x_ref, out_ref):
    st = st_ref[...]
    mean = st[0:1, :] / float(N)
    var = st[1:2, :] / float(N) - mean * mean
    o = (op_ref[...] - mean) / jnp.sqrt(var + EPS) * g_ref[...] + b_ref[...]
    o = jnp.maximum(o, 0.0)
    out_ref[...] = jnp.dot(o, _perm_mat(True),
                           preferred_element_type=jnp.float32) + x_ref[...]


def _tc_e(out_pre, ostats, g, b, x):
    return pl.pallas_call(
        _tc_e_body,
        grid=(NBLK_A,),
        in_specs=[
            pl.BlockSpec((BLK_A, P), lambda i: (i, 0)),
            pl.BlockSpec((2, P), lambda i: (0, 0)),
            pl.BlockSpec((1, P), lambda i: (0, 0)),
            pl.BlockSpec((1, P), lambda i: (0, 0)),
            pl.BlockSpec((BLK_A, P), lambda i: (i, 0)),
        ],
        out_specs=pl.BlockSpec((BLK_A, P), lambda i: (i, 0)),
        out_shape=jax.ShapeDtypeStruct((N, P), jnp.float32),
    )(out_pre, ostats, g, b, x)


def kernel(x, coords, neis_in, neis_out, W_q, gamma_q, beta_q, W_v, gamma_v,
           beta_v, W_pos, b_pos, W_mapqk, b_mapqk, gamma_out, beta_out):
    wq_all = jnp.transpose(W_q, (1, 0, 2)).reshape(P, K * V)
    nin_pad = jnp.pad(neis_in, ((0, 0), (0, N_PAD - N)))
    wm48 = jnp.pad(W_mapqk, ((0, 0), (0, QW - V)))

    nin_t = jnp.pad(nin_pad, ((0, KP - K), (0, 0))).T.reshape(-1)
    xv, xq, m, vstats = _tc_a(x, W_v, wq_all)
    q_pre = _sc_g1(xq.reshape(N * K, V), nin_t)
    m_pad = jnp.pad(m, ((0, N_PAD - N), (0, 0)))
    qmx = _tc_c(q_pre, m_pad, gamma_q.reshape(1, V), beta_q.reshape(1, V),
                wm48)
    v_f = _tc_c2(xv, vstats, gamma_v.reshape(1, P), beta_v.reshape(1, P))
    out_pre = _sc_attn(qmx, v_f, nin_t, b_mapqk)
    ostats = _tc_stats(out_pre)
    perm = (jnp.arange(P) % V) * (P // V) + jnp.arange(P) // V
    return _tc_e(out_pre, ostats, gamma_out[perm].reshape(1, P),
                 beta_out[perm].reshape(1, P), x)


# speedup vs baseline: 1.0032x; 1.0032x over previous
import functools

import jax
import jax.numpy as jnp
from jax import lax
from jax.experimental import pallas as pl
from jax.experimental.pallas import tpu as pltpu
from jax.experimental.pallas import tpu_sc as plsc

N = 10000
P = 256
V = 32
K = 27
N_PAD = 10240
NW = 32
RPW = N_PAD // NW
QW = 48
EPS = 1e-5

R_SLOW = 256
R_FAST = 384
SLOW_CORE = 1
B_FAST = 16 * R_SLOW

CH1 = 64
NC1 = RPW // CH1

KP = 28
GCH = 32
NGC = RPW // GCH
GRP = 4
NGRP = GCH // GRP
GIW = KP * GRP

BLK_A = 1000
NBLK_A = N // BLK_A
BLK_S = 1024
NBLK_S = N_PAD // BLK_S

_SC_MESH = plsc.VectorSubcoreMesh(core_axis_name="c", subcore_axis_name="s")
_SC_PARAMS = pltpu.CompilerParams(use_tc_tiling_on_sc=False)


def _tc_a_body(x_ref, wv_ref, wq_ref, xv_ref, xq_ref, m_ref, vstats_ref, acc):
    i = pl.program_id(0)
    xb = x_ref[...]
    xv = jnp.dot(xb, wv_ref[...], preferred_element_type=jnp.float32)
    xq_ref[...] = jnp.dot(xb, wq_ref[...], preferred_element_type=jnp.float32)
    xv_ref[...] = xv
    m_ref[...] = (jnp.sum(jnp.abs(xb), axis=1, keepdims=True) > 0.0).astype(
        jnp.float32)

    @pl.when(i == 0)
    def _():
        acc[...] = jnp.zeros_like(acc)

    s = jnp.sum(xv, axis=0, keepdims=True)
    ss = jnp.sum(xv * xv, axis=0, keepdims=True)
    acc[...] = acc[...] + jnp.concatenate([s, ss], axis=0)

    @pl.when(i == NBLK_A - 1)
    def _():
        vstats_ref[...] = acc[...]


def _tc_a(x, wv, wq_all):
    return pl.pallas_call(
        _tc_a_body,
        grid=(NBLK_A,),
        in_specs=[
            pl.BlockSpec((BLK_A, P), lambda i: (i, 0)),
            pl.BlockSpec((P, P), lambda i: (0, 0)),
            pl.BlockSpec((P, K * V), lambda i: (0, 0)),
        ],
        out_specs=[
            pl.BlockSpec((BLK_A, P), lambda i: (i, 0)),
            pl.BlockSpec((BLK_A, K * V), lambda i: (i, 0)),
            pl.BlockSpec((BLK_A, 1), lambda i: (i, 0)),
            pl.BlockSpec((2, P), lambda i: (0, 0)),
        ],
        out_shape=[
            jax.ShapeDtypeStruct((N, P), jnp.float32),
            jax.ShapeDtypeStruct((N, K * V), jnp.float32),
            jax.ShapeDtypeStruct((N, 1), jnp.float32),
            jax.ShapeDtypeStruct((2, P), jnp.float32),
        ],
        scratch_shapes=[pltpu.VMEM((2, P), jnp.float32)],
    )(x, wv, wq_all)


def _sc_g1_body(xq_hbm, nint_hbm, qpre_hbm, idx1, qg, acc_v, sem):
    slow = lax.axis_index("c") == SLOW_CORE
    s = lax.axis_index("s")
    base = jnp.where(slow, s * R_SLOW, B_FAST + s * R_FAST)
    nchunk = jnp.where(slow, R_SLOW // CH1, R_FAST // CH1)
    iota16 = lax.iota(jnp.int32, 16)

    def per_chunk(c, _):
        cb = base + c * CH1
        pltpu.sync_copy(nint_hbm.at[pl.ds(cb * KP, CH1 * KP)], idx1)

        def scale(t, _):
            sl = pl.ds(t * 16, 16)
            idx1[sl] = idx1[sl] * K + (iota16 + t * 16) % KP
            return 0

        lax.fori_loop(0, CH1 * KP // 16, scale, 0, unroll=4)

        def fire(g, _):
            pltpu.async_copy(xq_hbm.at[idx1.at[pl.ds(g * GIW, GIW)]],
                             qg.at[pl.ds(g * GIW, GIW)], sem)
            return 0

        lax.fori_loop(0, CH1 // GRP, fire, 0)

        def drain(g, _):
            pltpu.make_async_copy(xq_hbm.at[idx1.at[pl.ds(g * GIW, GIW)]],
                                  qg.at[pl.ds(g * GIW, GIW)], sem).wait()
            return 0

        lax.fori_loop(0, CH1 // GRP, drain, 0)

        def acc_r(r, _):
            for c2 in range(V // 16):
                sl = pl.ds(c2 * 16, 16)

                def acc_k(k, s):
                    return s + qg[r * KP + k, sl]

                acc_v[r, sl] = lax.fori_loop(1, K, acc_k, qg[r * KP, sl],
                                             unroll=2)
            return 0

        lax.fori_loop(0, CH1, acc_r, 0, unroll=2)
        pltpu.sync_copy(acc_v, qpre_hbm.at[pl.ds(cb, CH1)])
        return 0

    lax.fori_loop(0, nchunk, per_chunk, 0)


def _sc_g1(xq_flat, nin_t):
    f = functools.partial(
        pl.kernel,
        mesh=_SC_MESH,
        compiler_params=_SC_PARAMS,
        out_type=jax.ShapeDtypeStruct((N_PAD, V), jnp.float32),
        scratch_types=[
            pltpu.VMEM((CH1 * KP,), jnp.int32),
            pltpu.VMEM((CH1 * KP, V), jnp.float32),
            pltpu.VMEM((CH1, V), jnp.float32),
            pltpu.SemaphoreType.DMA,
        ],
    )(_sc_g1_body)
    return f(xq_flat, nin_t)


def _tc_c_body(qpre_ref, m_ref, g_ref, b_ref, wm_ref, qmx_ref):
    qp = qpre_ref[...]
    rows = lax.broadcasted_iota(jnp.int32, (N_PAD, 1), 0)
    valid = (rows < N).astype(jnp.float32)
    qv = qp * valid
    s = jnp.sum(qv, axis=0, keepdims=True)
    ss = jnp.sum(qv * qv, axis=0, keepdims=True)
    mean = s / float(N)
    var = ss / float(N) - mean * mean
    qf = (qp - mean) / jnp.sqrt(var + EPS) * g_ref[...] + b_ref[...]
    qf = jnp.maximum(qf, 0.0)
    qm48 = jnp.dot(qf, wm_ref[...], preferred_element_type=jnp.float32)
    mcol = (lax.broadcasted_iota(jnp.int32, (1, QW), 1) == V).astype(
        jnp.float32)
    qmx_ref[...] = qm48 + m_ref[...] * mcol


def _tc_c(q_pre, m_pad, g, b, wm48):
    return pl.pallas_call(
        _tc_c_body,
        out_shape=jax.ShapeDtypeStruct((N_PAD, QW), jnp.float32),
    )(q_pre, m_pad, g, b, wm48)


def _perm_mat(inverse):
    rows = lax.broadcasted_iota(jnp.int32, (P, P), 0)
    cols = lax.broadcasted_iota(jnp.int32, (P, P), 1)
    if inverse:
        return ((rows % V) * (P // V) + rows // V == cols).astype(jnp.float32)
    return (rows == (cols % V) * (P // V) + cols // V).astype(jnp.float32)


def _tc_c2_body(xv_ref, st_ref, g_ref, b_ref, vf_ref):
    st = st_ref[...]
    mean = st[0:1, :] / float(N)
    var = st[1:2, :] / float(N) - mean * mean
    vf = (xv_ref[...] - mean) / jnp.sqrt(var + EPS) * g_ref[...] + b_ref[...]
    vf = jnp.maximum(vf, 0.0)
    vf_ref[...] = jnp.dot(vf, _perm_mat(False),
                          preferred_element_type=jnp.float32)


def _tc_c2(xv, vstats, g, b):
    return pl.pallas_call(
        _tc_c2_body,
        grid=(NBLK_A,),
        in_specs=[
            pl.BlockSpec((BLK_A, P), lambda i: (i, 0)),
            pl.BlockSpec((2, P), lambda i: (0, 0)),
            pl.BlockSpec((1, P), lambda i: (0, 0)),
            pl.BlockSpec((1, P), lambda i: (0, 0)),
        ],
        out_specs=pl.BlockSpec((BLK_A, P), lambda i: (i, 0)),
        out_shape=jax.ShapeDtypeStruct((N, P), jnp.float32),
    )(xv, vstats, g, b)


def _sc_attn_body(qmx_hbm, vf_hbm, nint_hbm, bm_hbm, out_hbm,
                  qm_own, idx1, qxg, vg, outb, bm_v,
                  sem_q, sem_v0, sem_v1):
    slow = lax.axis_index("c") == SLOW_CORE
    s = lax.axis_index("s")
    base = jnp.where(slow, s * R_SLOW, B_FAST + s * R_FAST)
    nchunk = jnp.where(slow, R_SLOW // GCH, R_FAST // GCH)
    pltpu.sync_copy(bm_hbm, bm_v)

    def per_chunk(c, _):
        cb = base + c * GCH
        pltpu.sync_copy(qmx_hbm.at[pl.ds(cb, GCH)], qm_own)
        pltpu.sync_copy(nint_hbm.at[pl.ds(cb * KP, GCH * KP)], idx1)

        for g in range(NGRP):
            pltpu.async_copy(qmx_hbm.at[idx1.at[pl.ds(g * GIW, GIW)]],
                             qxg.at[pl.ds(g * GIW, GIW)], sem_q)
        def fire_vg(g, par):
            sem = sem_v0 if par == 0 else sem_v1
            for i in range(GIW // 16):
                pltpu.async_copy(
                    vf_hbm.at[idx1.at[pl.ds(g * GIW + i * 16, 16)]],
                    vg.at[par, pl.ds(i * 16, 16)], sem)

        def drain_vg(g, par):
            sem = sem_v0 if par == 0 else sem_v1
            for i in range(GIW // 16):
                pltpu.make_async_copy(
                    vf_hbm.at[idx1.at[pl.ds(g * GIW + i * 16, 16)]],
                    vg.at[par, pl.ds(i * 16, 16)], sem).wait()

        fire_vg(0, 0)
        fire_vg(1, 1)
        for g in range(NGRP):
            pltpu.make_async_copy(qmx_hbm.at[idx1.at[pl.ds(g * GIW, GIW)]],
                                  qxg.at[pl.ds(g * GIW, GIW)], sem_q).wait()

        def lg_r(r, _):
            def lg_k(k, _):
                p = r * KP + k
                mk = qxg[p, pl.ds(V, 16)][0]
                for c2 in range(V // 16):
                    sl = pl.ds(c2 * 16, 16)
                    qxg[p, sl] = ((qxg[p, sl] - qm_own[r, sl] * mk
                                   + bm_v[sl]) * mk)
                return 0

            lax.fori_loop(0, K, lg_k, 0, unroll=3)
            return 0

        lax.fori_loop(0, GCH, lg_r, 0)

        def sm_r(r, _):
            p0 = r * KP
            for c2 in range(V // 16):
                sl = pl.ds(c2 * 16, 16)

                def mxk(k, m):
                    return jnp.maximum(m, qxg[p0 + k, sl])

                mx = lax.fori_loop(1, K, mxk, qxg[p0, sl], unroll=2)

                def esk(k, s):
                    e = jnp.exp(qxg[p0 + k, sl] - mx)
                    qxg[p0 + k, sl] = e
                    return s + e

                s = lax.fori_loop(0, K, esk, jnp.zeros((16,), jnp.float32),
                                  unroll=3)
                rinv = 1.0 / s

                def nrm(k, _):
                    mk = qxg[p0 + k, pl.ds(V, 16)][0]
                    qxg[p0 + k, sl] = qxg[p0 + k, sl] * (rinv * mk)
                    return 0

                lax.fori_loop(0, K, nrm, 0, unroll=3)
            return 0

        lax.fori_loop(0, GCH, sm_r, 0)

        for g in range(NGRP):
            par = g % 2
            drain_vg(g, par)

            def row_j(j, _):
                r = g * GRP + j
                p0 = r * KP
                q0 = j * KP

                def k_acc(k, acc):
                    a0 = qxg[p0 + k, pl.ds(0, 16)]
                    a1 = qxg[p0 + k, pl.ds(16, 16)]
                    new = []
                    for cc in range(16):
                        row = vg[par, q0 + k, pl.ds(cc * 16, 16)]
                        new.append(acc[cc] + row * (a0 if cc % 2 == 0
                                                    else a1))
                    return tuple(new)

                acc = lax.fori_loop(
                    0, K, k_acc,
                    tuple(jnp.zeros((16,), jnp.float32) for _ in range(16)),
                    unroll=3)
                for cc in range(16):
                    outb[r, pl.ds(cc * 16, 16)] = acc[cc]
                return 0

            lax.fori_loop(0, GRP, row_j, 0)
            if g + 2 < NGRP:
                fire_vg(g + 2, par)

        pltpu.sync_copy(outb, out_hbm.at[pl.ds(cb, GCH)])
        return 0

    lax.fori_loop(0, nchunk, per_chunk, 0)


def _sc_attn(qmx, v_f, nin_t, bm):
    f = functools.partial(
        pl.kernel,
        mesh=_SC_MESH,
        compiler_params=_SC_PARAMS,
        out_type=jax.ShapeDtypeStruct((N_PAD, P), jnp.float32),
        scratch_types=[
            pltpu.VMEM((GCH, QW), jnp.float32),
            pltpu.VMEM((GCH * KP,), jnp.int32),
            pltpu.VMEM((GCH * KP, QW), jnp.float32),
            pltpu.VMEM((2, GIW, P), jnp.float32),
            pltpu.VMEM((GCH, P), jnp.float32),
            pltpu.VMEM((V,), jnp.float32),
            pltpu.SemaphoreType.DMA,
            pltpu.SemaphoreType.DMA,
            pltpu.SemaphoreType.DMA,
        ],
    )(_sc_attn_body)
    return f(qmx, v_f, nin_t, bm)


def _tc_stats_body(op_ref, st_ref, acc):
    i = pl.program_id(0)
    op = op_ref[...]
    rows = i * BLK_S + lax.broadcasted_iota(jnp.int32, (BLK_S, 1), 0)
    valid = (rows < N).astype(jnp.float32)
    ov = op * valid

    @pl.when(i == 0)
    def _():
        acc[...] = jnp.zeros_like(acc)

    s = jnp.sum(ov, axis=0, keepdims=True)
    ss = jnp.sum(ov * ov, axis=0, keepdims=True)
    acc[...] = acc[...] + jnp.concatenate([s, ss], axis=0)

    @pl.when(i == NBLK_S - 1)
    def _():
        st_ref[...] = acc[...]


def _tc_stats(out_pre):
    return pl.pallas_call(
        _tc_stats_body,
        grid=(NBLK_S,),
        in_specs=[pl.BlockSpec((BLK_S, P), lambda i: (i, 0))],
        out_specs=pl.BlockSpec((2, P), lambda i: (0, 0)),
        out_shape=jax.ShapeDtypeStruct((2, P), jnp.float32),
        scratch_shapes=[pltpu.VMEM((2, P), jnp.float32)],
    )(out_pre)


def _tc_e_body(op_ref, st_ref, g_ref, b_ref, x_ref, out_ref):
    st = st_ref[...]
    mean = st[0:1, :] / float(N)
    var = st[1:2, :] / float(N) - mean * mean
    o = (op_ref[...] - mean) / jnp.sqrt(var + EPS) * g_ref[...] + b_ref[...]
    o = jnp.maximum(o, 0.0)
    out_ref[...] = jnp.dot(o, _perm_mat(True),
                           preferred_element_type=jnp.float32) + x_ref[...]


def _tc_e(out_pre, ostats, g, b, x):
    return pl.pallas_call(
        _tc_e_body,
        grid=(NBLK_A,),
        in_specs=[
            pl.BlockSpec((BLK_A, P), lambda i: (i, 0)),
            pl.BlockSpec((2, P), lambda i: (0, 0)),
            pl.BlockSpec((1, P), lambda i: (0, 0)),
            pl.BlockSpec((1, P), lambda i: (0, 0)),
            pl.BlockSpec((BLK_A, P), lambda i: (i, 0)),
        ],
        out_specs=pl.BlockSpec((BLK_A, P), lambda i: (i, 0)),
        out_shape=jax.ShapeDtypeStruct((N, P), jnp.float32),
    )(out_pre, ostats, g, b, x)


def kernel(x, coords, neis_in, neis_out, W_q, gamma_q, beta_q, W_v, gamma_v,
           beta_v, W_pos, b_pos, W_mapqk, b_mapqk, gamma_out, beta_out):
    wq_all = jnp.transpose(W_q, (1, 0, 2)).reshape(P, K * V)
    nin_pad = jnp.pad(neis_in, ((0, 0), (0, N_PAD - N)))
    wm48 = jnp.pad(W_mapqk, ((0, 0), (0, QW - V)))

    nin_t = jnp.pad(nin_pad, ((0, KP - K), (0, 0))).T.reshape(-1)
    xv, xq, m, vstats = _tc_a(x, W_v, wq_all)
    q_pre = _sc_g1(xq.reshape(N * K, V), nin_t)
    m_pad = jnp.pad(m, ((0, N_PAD - N), (0, 0)))
    qmx = _tc_c(q_pre, m_pad, gamma_q.reshape(1, V), beta_q.reshape(1, V),
                wm48)
    v_f = _tc_c2(xv, vstats, gamma_v.reshape(1, P), beta_v.reshape(1, P))
    out_pre = _sc_attn(qmx, v_f, nin_t, b_mapqk)
    ostats = _tc_stats(out_pre)
    perm = (jnp.arange(P) % V) * (P // V) + jnp.arange(P) // V
    return _tc_e(out_pre, ostats, gamma_out[perm].reshape(1, P),
                 beta_out[perm].reshape(1, P), x)


# speedup vs baseline: 1.2323x; 1.2284x over previous
import functools

import jax
import jax.numpy as jnp
from jax import lax
from jax.experimental import pallas as pl
from jax.experimental.pallas import tpu as pltpu
from jax.experimental.pallas import tpu_sc as plsc

N = 10000
P = 256
V = 32
K = 27
N_PAD = 10240
NW = 32
RPW = N_PAD // NW
QW = 48
EPS = 1e-5

KP = 28
CH1 = 64
NC1 = RPW // CH1
G1W = KP * 4

GCH = 32
NGC = RPW // GCH
GRP = 4
NGRP = GCH // GRP
GIW = K * GRP

BLK_A = 1000
NBLK_A = N // BLK_A
BLK_S = 1024
NBLK_S = N_PAD // BLK_S

_SC_MESH = plsc.VectorSubcoreMesh(core_axis_name="c", subcore_axis_name="s")
_SC_PARAMS = pltpu.CompilerParams(use_tc_tiling_on_sc=False)


def _tc_a_body(x_ref, wv_ref, wq_ref, xv_ref, xq_ref, m_ref, vstats_ref, acc):
    i = pl.program_id(0)
    xb = x_ref[...]
    xv = jnp.dot(xb, wv_ref[...], preferred_element_type=jnp.float32)
    xq_ref[...] = jnp.dot(xb, wq_ref[...], preferred_element_type=jnp.float32)
    xv_ref[...] = xv
    m_ref[...] = (jnp.sum(jnp.abs(xb), axis=1, keepdims=True) > 0.0).astype(
        jnp.float32)

    @pl.when(i == 0)
    def _():
        acc[...] = jnp.zeros_like(acc)

    s = jnp.sum(xv, axis=0, keepdims=True)
    ss = jnp.sum(xv * xv, axis=0, keepdims=True)
    acc[...] = acc[...] + jnp.concatenate([s, ss], axis=0)

    @pl.when(i == NBLK_A - 1)
    def _():
        vstats_ref[...] = acc[...]


def _tc_a(x, wv, wq_all):
    return pl.pallas_call(
        _tc_a_body,
        grid=(NBLK_A,),
        in_specs=[
            pl.BlockSpec((BLK_A, P), lambda i: (i, 0)),
            pl.BlockSpec((P, P), lambda i: (0, 0)),
            pl.BlockSpec((P, K * V), lambda i: (0, 0)),
        ],
        out_specs=[
            pl.BlockSpec((BLK_A, P), lambda i: (i, 0)),
            pl.BlockSpec((BLK_A, K * V), lambda i: (i, 0)),
            pl.BlockSpec((BLK_A, 1), lambda i: (i, 0)),
            pl.BlockSpec((2, P), lambda i: (0, 0)),
        ],
        out_shape=[
            jax.ShapeDtypeStruct((N, P), jnp.float32),
            jax.ShapeDtypeStruct((N, K * V), jnp.float32),
            jax.ShapeDtypeStruct((N, 1), jnp.float32),
            jax.ShapeDtypeStruct((2, P), jnp.float32),
        ],
        scratch_shapes=[pltpu.VMEM((2, P), jnp.float32)],
    )(x, wv, wq_all)


def _sc_g1_body(xq_hbm, nint_hbm, qpre_hbm, idx1, qg, acc_v, sem):
    wid = lax.axis_index("c") * 16 + lax.axis_index("s")
    base = wid * RPW
    iota16 = lax.iota(jnp.int32, 16)

    def per_chunk(c, _):
        cb = base + c * CH1
        pltpu.sync_copy(nint_hbm.at[pl.ds(cb * KP, CH1 * KP)], idx1)

        def scale(t, _):
            sl = pl.ds(t * 16, 16)
            idx1[sl] = idx1[sl] * K + (iota16 + t * 16) % KP
            return 0

        lax.fori_loop(0, CH1 * KP // 16, scale, 0, unroll=4)

        def fire(g, _):
            pltpu.async_copy(xq_hbm.at[idx1.at[pl.ds(g * G1W, G1W)]],
                             qg.at[pl.ds(g * G1W, G1W)], sem)
            return 0

        lax.fori_loop(0, CH1 * KP // G1W, fire, 0)

        def drain(g, _):
            pltpu.make_async_copy(xq_hbm.at[idx1.at[pl.ds(g * G1W, G1W)]],
                                  qg.at[pl.ds(g * G1W, G1W)], sem).wait()
            return 0

        lax.fori_loop(0, CH1 * KP // G1W, drain, 0)

        def acc_r(r, _):
            for c2 in range(V // 16):
                sl = pl.ds(c2 * 16, 16)

                def acc_k(k, s):
                    return s + qg[r * KP + k, sl]

                acc_v[r, sl] = lax.fori_loop(1, K, acc_k, qg[r * KP, sl],
                                             unroll=2)
            return 0

        lax.fori_loop(0, CH1, acc_r, 0, unroll=2)
        pltpu.sync_copy(acc_v, qpre_hbm.at[pl.ds(cb, CH1)])
        return 0

    lax.fori_loop(0, NC1, per_chunk, 0)


def _sc_g1(xq_flat, nin_t):
    f = functools.partial(
        pl.kernel,
        mesh=_SC_MESH,
        compiler_params=_SC_PARAMS,
        out_type=jax.ShapeDtypeStruct((N_PAD, V), jnp.float32),
        scratch_types=[
            pltpu.VMEM((CH1 * KP,), jnp.int32),
            pltpu.VMEM((CH1 * KP, V), jnp.float32),
            pltpu.VMEM((CH1, V), jnp.float32),
            pltpu.SemaphoreType.DMA,
        ],
    )(_sc_g1_body)
    return f(xq_flat, nin_t)


def _tc_c_body(qpre_ref, m_ref, g_ref, b_ref, wm_ref, qmx_ref):
    qp = qpre_ref[...]
    rows = lax.broadcasted_iota(jnp.int32, (N_PAD, 1), 0)
    valid = (rows < N).astype(jnp.float32)
    qv = qp * valid
    s = jnp.sum(qv, axis=0, keepdims=True)
    ss = jnp.sum(qv * qv, axis=0, keepdims=True)
    mean = s / float(N)
    var = ss / float(N) - mean * mean
    qf = (qp - mean) / jnp.sqrt(var + EPS) * g_ref[...] + b_ref[...]
    qf = jnp.maximum(qf, 0.0)
    qm48 = jnp.dot(qf, wm_ref[...], preferred_element_type=jnp.float32)
    mcol = (lax.broadcasted_iota(jnp.int32, (1, QW), 1) == V).astype(
        jnp.float32)
    qmx_ref[...] = qm48 + m_ref[...] * mcol


def _tc_c(q_pre, m_pad, g, b, wm48):
    return pl.pallas_call(
        _tc_c_body,
        out_shape=jax.ShapeDtypeStruct((N_PAD, QW), jnp.float32),
    )(q_pre, m_pad, g, b, wm48)


def _perm_mat(inverse):
    rows = lax.broadcasted_iota(jnp.int32, (P, P), 0)
    cols = lax.broadcasted_iota(jnp.int32, (P, P), 1)
    if inverse:
        return ((rows % V) * (P // V) + rows // V == cols).astype(jnp.float32)
    return (rows == (cols % V) * (P // V) + cols // V).astype(jnp.float32)


def _tc_c2_body(xv_ref, st_ref, g_ref, b_ref, vf_ref):
    st = st_ref[...]
    mean = st[0:1, :] / float(N)
    var = st[1:2, :] / float(N) - mean * mean
    vf = (xv_ref[...] - mean) / jnp.sqrt(var + EPS) * g_ref[...] + b_ref[...]
    vf = jnp.maximum(vf, 0.0)
    vf_ref[...] = jnp.dot(vf, _perm_mat(False),
                          preferred_element_type=jnp.float32)


def _tc_c2(xv, vstats, g, b):
    return pl.pallas_call(
        _tc_c2_body,
        grid=(NBLK_A,),
        in_specs=[
            pl.BlockSpec((BLK_A, P), lambda i: (i, 0)),
            pl.BlockSpec((2, P), lambda i: (0, 0)),
            pl.BlockSpec((1, P), lambda i: (0, 0)),
            pl.BlockSpec((1, P), lambda i: (0, 0)),
        ],
        out_specs=pl.BlockSpec((BLK_A, P), lambda i: (i, 0)),
        out_shape=jax.ShapeDtypeStruct((N, P), jnp.float32),
    )(xv, vstats, g, b)


def _sc_attn_body(qmx_hbm, vf_hbm, nin_hbm, bm_hbm, out_hbm,
                  qm_own, idx2d, qxg, vg, outb, bm_v,
                  sem_i, sem_q, sem_v0, sem_v1):
    wid = lax.axis_index("c") * 16 + lax.axis_index("s")
    base = wid * RPW
    pltpu.sync_copy(bm_hbm, bm_v)

    def per_chunk(c, _):
        cb = base + c * GCH
        pltpu.sync_copy(qmx_hbm.at[pl.ds(cb, GCH)], qm_own)

        def fire_i(k, _):
            pltpu.async_copy(nin_hbm.at[pl.ds(k * N_PAD + cb, GCH)],
                             idx2d.at[k], sem_i)
            return 0

        lax.fori_loop(0, K, fire_i, 0)

        def drain_i(k, _):
            pltpu.make_async_copy(nin_hbm.at[pl.ds(k * N_PAD + cb, GCH)],
                                  idx2d.at[k], sem_i).wait()
            return 0

        lax.fori_loop(0, K, drain_i, 0)

        def fire_q(k, _):
            pltpu.async_copy(qmx_hbm.at[idx2d.at[k]], qxg.at[k], sem_q)
            return 0

        lax.fori_loop(0, K, fire_q, 0)

        def drain_q(k, _):
            pltpu.make_async_copy(qmx_hbm.at[idx2d.at[k]], qxg.at[k],
                                  sem_q).wait()
            return 0

        lax.fori_loop(0, K, drain_q, 0)

        def lg_k(k, _):
            def lg_r(r, _):
                mk = qxg[k, r, pl.ds(V, 16)][0]
                for c2 in range(V // 16):
                    sl = pl.ds(c2 * 16, 16)
                    qxg[k, r, sl] = ((qxg[k, r, sl] - qm_own[r, sl] * mk
                                      + bm_v[sl]) * mk)
                return 0

            lax.fori_loop(0, GCH, lg_r, 0, unroll=4)
            return 0

        lax.fori_loop(0, K, lg_k, 0)

        def sm_r(r, _):
            for c2 in range(V // 16):
                sl = pl.ds(c2 * 16, 16)

                def mxk(k, m):
                    return jnp.maximum(m, qxg[k, r, sl])

                mx = lax.fori_loop(1, K, mxk, qxg[0, r, sl], unroll=2)

                def esk(k, s):
                    e = jnp.exp(qxg[k, r, sl] - mx)
                    qxg[k, r, sl] = e
                    return s + e

                s = lax.fori_loop(0, K, esk, jnp.zeros((16,), jnp.float32),
                                  unroll=3)
                rinv = 1.0 / s

                def nrm(k, _):
                    mk = qxg[k, r, pl.ds(V, 16)][0]
                    qxg[k, r, sl] = qxg[k, r, sl] * (rinv * mk)
                    return 0

                lax.fori_loop(0, K, nrm, 0, unroll=3)
            return 0

        lax.fori_loop(0, GCH, sm_r, 0)

        def fire_v(g, par):
            sem = sem_v0 if par == 0 else sem_v1

            def fk(k, _):
                pltpu.async_copy(
                    vf_hbm.at[idx2d.at[k, pl.ds(g * GRP, GRP)]],
                    vg.at[par, k], sem)
                return 0

            lax.fori_loop(0, K, fk, 0)

        def drain_v(g, par):
            sem = sem_v0 if par == 0 else sem_v1

            def dk(k, _):
                pltpu.make_async_copy(
                    vf_hbm.at[idx2d.at[k, pl.ds(g * GRP, GRP)]],
                    vg.at[par, k], sem).wait()
                return 0

            lax.fori_loop(0, K, dk, 0)

        fire_v(0, 0)
        for g in range(NGRP):
            par = g % 2
            if g + 1 < NGRP:
                fire_v(g + 1, (g + 1) % 2)
            drain_v(g, par)

            def row_j(j, _):
                r = g * GRP + j

                def k_acc(k, acc):
                    a0 = qxg[k, r, pl.ds(0, 16)]
                    a1 = qxg[k, r, pl.ds(16, 16)]
                    new = []
                    for cc in range(16):
                        row = vg[par, k, j, pl.ds(cc * 16, 16)]
                        new.append(acc[cc] + row * (a0 if cc % 2 == 0
                                                    else a1))
                    return tuple(new)

                acc = lax.fori_loop(
                    0, K, k_acc,
                    tuple(jnp.zeros((16,), jnp.float32) for _ in range(16)),
                    unroll=3)
                for c8 in range(16):
                    outb[r, pl.ds(c8 * 16, 16)] = acc[c8]
                return 0

            lax.fori_loop(0, GRP, row_j, 0)

        pltpu.sync_copy(outb, out_hbm.at[pl.ds(cb, GCH)])
        return 0

    lax.fori_loop(0, NGC, per_chunk, 0)


def _sc_attn(qmx, v_f, nin_flat, bm):
    f = functools.partial(
        pl.kernel,
        mesh=_SC_MESH,
        compiler_params=_SC_PARAMS,
        out_type=jax.ShapeDtypeStruct((N_PAD, P), jnp.float32),
        scratch_types=[
            pltpu.VMEM((GCH, QW), jnp.float32),
            pltpu.VMEM((K, GCH), jnp.int32),
            pltpu.VMEM((K, GCH, QW), jnp.float32),
            pltpu.VMEM((2, K, GRP, P), jnp.float32),
            pltpu.VMEM((GCH, P), jnp.float32),
            pltpu.VMEM((V,), jnp.float32),
            pltpu.SemaphoreType.DMA,
            pltpu.SemaphoreType.DMA,
            pltpu.SemaphoreType.DMA,
            pltpu.SemaphoreType.DMA,
        ],
    )(_sc_attn_body)
    return f(qmx, v_f, nin_flat, bm)


def _tc_stats_body(op_ref, st_ref, acc):
    i = pl.program_id(0)
    op = op_ref[...]
    rows = i * BLK_S + lax.broadcasted_iota(jnp.int32, (BLK_S, 1), 0)
    valid = (rows < N).astype(jnp.float32)
    ov = op * valid

    @pl.when(i == 0)
    def _():
        acc[...] = jnp.zeros_like(acc)

    s = jnp.sum(ov, axis=0, keepdims=True)
    ss = jnp.sum(ov * ov, axis=0, keepdims=True)
    acc[...] = acc[...] + jnp.concatenate([s, ss], axis=0)

    @pl.when(i == NBLK_S - 1)
    def _():
        st_ref[...] = acc[...]


def _tc_stats(out_pre):
    return pl.pallas_call(
        _tc_stats_body,
        grid=(NBLK_S,),
        in_specs=[pl.BlockSpec((BLK_S, P), lambda i: (i, 0))],
        out_specs=pl.BlockSpec((2, P), lambda i: (0, 0)),
        out_shape=jax.ShapeDtypeStruct((2, P), jnp.float32),
        scratch_shapes=[pltpu.VMEM((2, P), jnp.float32)],
    )(out_pre)


def _tc_e_body(op_ref, st_ref, g_ref, b_ref, x_ref, out_ref):
    st = st_ref[...]
    mean = st[0:1, :] / float(N)
    var = st[1:2, :] / float(N) - mean * mean
    o = (op_ref[...] - mean) / jnp.sqrt(var + EPS) * g_ref[...] + b_ref[...]
    o = jnp.maximum(o, 0.0)
    out_ref[...] = jnp.dot(o, _perm_mat(True),
                           preferred_element_type=jnp.float32) + x_ref[...]


def _tc_e(out_pre, ostats, g, b, x):
    return pl.pallas_call(
        _tc_e_body,
        grid=(NBLK_A,),
        in_specs=[
            pl.BlockSpec((BLK_A, P), lambda i: (i, 0)),
            pl.BlockSpec((2, P), lambda i: (0, 0)),
            pl.BlockSpec((1, P), lambda i: (0, 0)),
            pl.BlockSpec((1, P), lambda i: (0, 0)),
            pl.BlockSpec((BLK_A, P), lambda i: (i, 0)),
        ],
        out_specs=pl.BlockSpec((BLK_A, P), lambda i: (i, 0)),
        out_shape=jax.ShapeDtypeStruct((N, P), jnp.float32),
    )(out_pre, ostats, g, b, x)


def kernel(x, coords, neis_in, neis_out, W_q, gamma_q, beta_q, W_v, gamma_v,
           beta_v, W_pos, b_pos, W_mapqk, b_mapqk, gamma_out, beta_out):
    wq_all = jnp.transpose(W_q, (1, 0, 2)).reshape(P, K * V)
    nin_pad = jnp.pad(neis_in, ((0, 0), (0, N_PAD - N)))
    wm48 = jnp.pad(W_mapqk, ((0, 0), (0, QW - V)))

    nin_flat = nin_pad.reshape(-1)
    nin_t = jnp.pad(nin_pad, ((0, KP - K), (0, 0))).T.reshape(-1)
    xv, xq, m, vstats = _tc_a(x, W_v, wq_all)
    q_pre = _sc_g1(xq.reshape(N * K, V), nin_t)
    m_pad = jnp.pad(m, ((0, N_PAD - N), (0, 0)))
    qmx = _tc_c(q_pre, m_pad, gamma_q.reshape(1, V), beta_q.reshape(1, V),
                wm48)
    v_f = _tc_c2(xv, vstats, gamma_v.reshape(1, P), beta_v.reshape(1, P))
    out_pre = _sc_attn(qmx, v_f, nin_flat, b_mapqk)
    ostats = _tc_stats(out_pre)
    perm = (jnp.arange(P) % V) * (P // V) + jnp.arange(P) // V
    return _tc_e(out_pre, ostats, gamma_out[perm].reshape(1, P),
                 beta_out[perm].reshape(1, P), x)
